# Initial kernel scaffold; baseline (speedup 1.0000x reference)
import jax
import jax.numpy as jnp
from jax.experimental import pallas as pl


def kernel(inputs, edge_index, states, actions, W_self1, W_neigh1, b1, W_self2, W_neigh2, b2, fcW, fcb):
    raise NotImplementedError("write your pallas kernel here")


# speedup vs baseline: 5.9065x; 5.9065x over previous
import functools

import jax
import jax.numpy as jnp
from jax import lax
from jax.experimental import pallas as pl
from jax.experimental.pallas import tpu as pltpu
from jax.experimental.pallas import tpu_sc as plsc

N = 10000
E = 320000
D = 128

NC = 2
NS = 16
NW = NC * NS
LANES = 16

EPT = E // NW
KCH = 125
NCH = EPT // KCH

NPAD = 10240
RPT = NPAD // NS
ZR = 128
DW = 16

S2P = 48


def _fill_const(ref, rows, cols, val):
    v = jnp.full((LANES,), val, jnp.float32)

    def body(r, carry):
        for c in range(cols // LANES):
            ref[r, pl.ds(c * LANES, LANES)] = v
        return carry

    lax.fori_loop(0, rows, body, 0)


def _sc_mesh():
    return plsc.VectorSubcoreMesh(core_axis_name="c", subcore_axis_name="s",
                                  num_cores=NC, num_subcores=NS)


def _sc_degree(dst3):
    out_type = [
        jax.ShapeDtypeStruct((NPAD, DW), jnp.float32),
        jax.ShapeDtypeStruct((NPAD, DW), jnp.float32),
    ]
    scratch = [
        pltpu.VMEM((NCH, KCH), jnp.int32),
        pltpu.VMEM((ZR, DW), jnp.float32),
        pltpu.VMEM_SHARED((NPAD, DW), jnp.float32),
        pltpu.SemaphoreType.DMA,
    ]

    @functools.partial(
        pl.kernel, out_type=out_type, mesh=_sc_mesh(),
        scratch_types=scratch,
        compiler_params=pltpu.CompilerParams(use_tc_tiling_on_sc=False))
    def deg_kernel(dst_ref, degA, degB, dbuf, dzbuf, degS, sem):
        c = lax.axis_index("c")
        s = lax.axis_index("s")
        w = c * NS + s

        _fill_const(dzbuf, ZR, DW, 0.0)
        for blk in range(RPT // ZR):
            pltpu.sync_copy(dzbuf, degS.at[pl.ds(s * RPT + blk * ZR, ZR)])

        plsc.subcore_barrier()

        pltpu.sync_copy(dst_ref.at[w], dbuf)
        _fill_const(dzbuf, KCH, DW, 1.0)

        def edge_body(j, carry):
            pltpu.sync_copy(dzbuf.at[pl.ds(0, KCH)],
                            degS.at[dbuf.at[j]], add=True)
            return carry

        lax.fori_loop(0, NCH, edge_body, 0)

        plsc.subcore_barrier()

        row0 = s * RPT

        @pl.when(c == 0)
        def _():
            pltpu.sync_copy(degS.at[pl.ds(row0, RPT)],
                            degA.at[pl.ds(row0, RPT)])

        @pl.when(c == 1)
        def _():
            pltpu.sync_copy(degS.at[pl.ds(row0, RPT)],
                            degB.at[pl.ds(row0, RPT)])

    return deg_kernel(dst3)


DH = D // 2


def _sc_aggregate(xh_hbm, src3, dst3):
    out_type = [
        jax.ShapeDtypeStruct((NPAD, DH), jnp.float32),
        jax.ShapeDtypeStruct((NPAD, DH), jnp.float32),
    ]
    scratch = [
        pltpu.VMEM((NCH, KCH), jnp.int32),
        pltpu.VMEM((NCH, KCH), jnp.int32),
        pltpu.VMEM((KCH, DH), jnp.float32),
        pltpu.VMEM((ZR, DH), jnp.float32),
        pltpu.VMEM_SHARED((NPAD, DH), jnp.float32),
        pltpu.SemaphoreType.DMA,
    ]

    @functools.partial(
        pl.kernel, out_type=out_type, mesh=_sc_mesh(),
        scratch_types=scratch,
        compiler_params=pltpu.CompilerParams(use_tc_tiling_on_sc=False))
    def agg_kernel(x_ref, src_ref, dst_ref, aggA, aggB,
                   sbuf, dbuf, rbuf, zbuf, aggS, sem):
        c = lax.axis_index("c")
        s = lax.axis_index("s")
        w = c * NS + s

        _fill_const(zbuf, ZR, DH, 0.0)
        for blk in range(RPT // ZR):
            pltpu.sync_copy(zbuf, aggS.at[pl.ds(s * RPT + blk * ZR, ZR)])

        plsc.subcore_barrier()

        pltpu.sync_copy(src_ref.at[w], sbuf)
        pltpu.sync_copy(dst_ref.at[w], dbuf)

        def edge_body(j, carry):
            pltpu.async_copy(x_ref.at[sbuf.at[j]], rbuf, sem).wait()
            pltpu.sync_copy(rbuf, aggS.at[dbuf.at[j]], add=True)
            return carry

        lax.fori_loop(0, NCH, edge_body, 0)

        plsc.subcore_barrier()

        row0 = s * RPT

        @pl.when(c == 0)
        def _():
            pltpu.sync_copy(aggS.at[pl.ds(row0, RPT)],
                            aggA.at[pl.ds(row0, RPT)])

        @pl.when(c == 1)
        def _():
            pltpu.sync_copy(aggS.at[pl.ds(row0, RPT)],
                            aggB.at[pl.ds(row0, RPT)])

    return agg_kernel(xh_hbm, src3, dst3)


def _tc_layer(xp, a0A, a0B, a1A, a1B, degA, degB, Ws, Wn, b):
    BR = 256
    nb = NPAD // BR

    def body(x_ref, a0A_ref, a0B_ref, a1A_ref, a1B_ref, dA_ref, dB_ref,
             ws_ref, wn_ref, b_ref, o_ref):
        deg = jnp.maximum(dA_ref[...][:, :1] + dB_ref[...][:, :1], 1.0)
        mean = jnp.concatenate(
            [(a0A_ref[...] + a0B_ref[...]) / deg,
             (a1A_ref[...] + a1B_ref[...]) / deg], axis=1)
        h = lax.dot_general(x_ref[...], ws_ref[...],
                            (((1,), (1,)), ((), ())),
                            preferred_element_type=jnp.float32)
        h = h + lax.dot_general(mean, wn_ref[...],
                                (((1,), (1,)), ((), ())),
                                preferred_element_type=jnp.float32)
        o_ref[...] = jnp.maximum(h + b_ref[...], 0.0)

    return pl.pallas_call(
        body,
        grid=(nb,),
        in_specs=[
            pl.BlockSpec((BR, D), lambda i: (i, 0)),
            pl.BlockSpec((BR, DH), lambda i: (i, 0)),
            pl.BlockSpec((BR, DH), lambda i: (i, 0)),
            pl.BlockSpec((BR, DH), lambda i: (i, 0)),
            pl.BlockSpec((BR, DH), lambda i: (i, 0)),
            pl.BlockSpec((BR, DW), lambda i: (i, 0)),
            pl.BlockSpec((BR, DW), lambda i: (i, 0)),
            pl.BlockSpec((D, D), lambda i: (0, 0)),
            pl.BlockSpec((D, D), lambda i: (0, 0)),
            pl.BlockSpec((1, D), lambda i: (0, 0)),
        ],
        out_specs=pl.BlockSpec((BR, D), lambda i: (i, 0)),
        out_shape=jax.ShapeDtypeStruct((NPAD, D), jnp.float32),
    )(xp, a0A, a0B, a1A, a1B, degA, degB, Ws, Wn, b)


def _tc_head(h1g, a2g, dg, Ws, Wn, b, fcW, fcb):
    def body(xg_ref, ag_ref, dg_ref, ws_ref, wn_ref, b_ref,
             fw_ref, fb_ref, o_ref):
        mean = ag_ref[...] / jnp.maximum(dg_ref[...], 1.0)
        h = lax.dot_general(xg_ref[...], ws_ref[...],
                            (((1,), (1,)), ((), ())),
                            preferred_element_type=jnp.float32)
        h = h + lax.dot_general(mean, wn_ref[...],
                                (((1,), (1,)), ((), ())),
                                preferred_element_type=jnp.float32)
        h = jnp.maximum(h + b_ref[...], 0.0)
        rows = lax.broadcasted_iota(jnp.int32, (S2P, D), 0)
        smax = jnp.max(jnp.where(rows < 32, h, -jnp.inf), axis=0,
                       keepdims=True)
        arow = h[32:33, :]
        fw = fw_ref[...]
        out = (jnp.sum(smax * fw[:, :D], axis=1, keepdims=True)
               + jnp.sum(arow * fw[:, D:], axis=1, keepdims=True)
               + fb_ref[...])
        o_ref[...] = out

    return pl.pallas_call(
        body,
        in_specs=[pl.BlockSpec(a.shape, lambda: (0, 0))
                  for a in (h1g, a2g, dg, Ws, Wn, b, fcW, fcb)],
        out_specs=pl.BlockSpec((1, 1), lambda: (0, 0)),
        out_shape=jax.ShapeDtypeStruct((1, 1), jnp.float32),
    )(h1g, a2g, dg, Ws, Wn, b, fcW, fcb)


def kernel(inputs, edge_index, states, actions, W_self1, W_neigh1, b1,
           W_self2, W_neigh2, b2, fcW, fcb):
    x = inputs.astype(jnp.float32)
    src = edge_index[0]
    dst = edge_index[1]

    src3 = src.reshape(NW, NCH, KCH)
    dst3 = dst.reshape(NW, NCH, KCH)

    s2idx = jnp.zeros((S2P,), jnp.int32).at[:32].set(states).at[32].set(
        actions[0])

    xp = jnp.pad(x, ((0, NPAD - N), (0, 0)))
    b1r = b1.reshape(1, D)
    b2r = b2.reshape(1, D)
    fcbr = fcb.reshape(1, 1)

    degA, degB = _sc_degree(dst3)
    a0A, a0B = _sc_aggregate(xp[:, :DH], src3, dst3)
    a1A, a1B = _sc_aggregate(xp[:, DH:], src3, dst3)
    h1 = _tc_layer(xp, a0A, a0B, a1A, a1B, degA, degB,
                   W_self1, W_neigh1, b1r)
    b0A, b0B = _sc_aggregate(h1[:, :DH], src3, dst3)
    b1A, b1B = _sc_aggregate(h1[:, DH:], src3, dst3)

    h1g = jnp.take(h1, s2idx, axis=0)
    a2g = jnp.concatenate(
        [jnp.take(b0A, s2idx, axis=0) + jnp.take(b0B, s2idx, axis=0),
         jnp.take(b1A, s2idx, axis=0) + jnp.take(b1B, s2idx, axis=0)],
        axis=1)
    dg = (jnp.take(degA[:, 0], s2idx) + jnp.take(degB[:, 0], s2idx))
    dg = dg.reshape(S2P, 1)

    out = _tc_head(h1g, a2g, dg, W_self2, W_neigh2, b2r, fcW, fcbr)
    return out
